# baseline (speedup 1.0000x reference)
import jax
import jax.numpy as jnp

from jax import lax
from jax.experimental import pallas as pl
from jax.experimental.pallas import tpu as pltpu

_KS = 4
_ST = 2
_PD = 1
_EPS = 1e-5
_B = 4


def kernel(x, weight, bias, gamma, beta):
    del bias
    N, C, H, W = x.shape
    Cout = weight.shape[0]
    Hout = (H + 2 * _PD - _KS) // _ST + 1
    Wout = (W + 2 * _PD - _KS) // _ST + 1
    Ha, Wa = H // 2, W // 2
    Mc = Hout * Wout
    C4 = 4 * C
    count = N * Mc
    G = N // _B

    xs = x.astype(jnp.bfloat16).reshape(N, C, Ha, 2, Wa, 2)
    xs = xs.transpose(0, 2, 4, 3, 5, 1).reshape(N, Ha * Wa, C4)

    kh_of = {(1, -1): 0, (0, 0): 1, (1, 0): 2, (0, 1): 3}
    wb = weight.astype(jnp.bfloat16)
    zero = jnp.zeros((C, Cout), jnp.bfloat16)
    row_blocks = []
    for pa in (0, 1):
        for pb in (0, 1):
            row = []
            for u in (-1, 0, 1):
                for v in (-1, 0, 1):
                    if (pa, u) in kh_of and (pb, v) in kh_of:
                        kh, kw = kh_of[(pa, u)], kh_of[(pb, v)]
                        row.append(wb[:, :, kh, kw].T)
                    else:
                        row.append(zero)
            row_blocks.append(jnp.concatenate(row, axis=1))
    wt = jnp.concatenate(row_blocks, axis=0)

    def conv_kernel(xs_ref, w_ref, y_ref, sum_ref, ssq_ref):
        t = jnp.dot(xs_ref[0], w_ref[...], preferred_element_type=jnp.float32)
        tp = jnp.pad(t, ((Wa + 1, Wa + 1), (0, 0)))
        jj = lax.broadcasted_iota(jnp.int32, (Mc, 1), 0) % Wout
        mask_m = (jj != 0).astype(jnp.float32)
        mask_p = (jj != Wout - 1).astype(jnp.float32)
        y = None
        for u in (-1, 0, 1):
            for v in (-1, 0, 1):
                tau = (u + 1) * 3 + (v + 1)
                base = (Wa + 1) + u * Wa + v
                term = tp[base:base + Mc, tau * Cout:(tau + 1) * Cout]
                if v == -1:
                    term = term * mask_m
                elif v == 1:
                    term = term * mask_p
                y = term if y is None else y + term
        sum_ref[0] = jnp.sum(y, axis=0, keepdims=True)
        ssq_ref[0] = jnp.sum(y * y, axis=0, keepdims=True)
        y_ref[0] = y.astype(jnp.bfloat16)

    y, sums, ssqs = pl.pallas_call(
        conv_kernel,
        out_shape=(jax.ShapeDtypeStruct((N, Mc, Cout), jnp.bfloat16),
                   jax.ShapeDtypeStruct((N, 1, Cout), jnp.float32),
                   jax.ShapeDtypeStruct((N, 1, Cout), jnp.float32)),
        grid=(N,),
        in_specs=[pl.BlockSpec((1, Ha * Wa, C4), lambda n: (n, 0, 0)),
                  pl.BlockSpec((C4, 9 * Cout), lambda n: (0, 0))],
        out_specs=(pl.BlockSpec((1, Mc, Cout), lambda n: (n, 0, 0)),
                   pl.BlockSpec((1, 1, Cout), lambda n: (n, 0, 0)),
                   pl.BlockSpec((1, 1, Cout), lambda n: (n, 0, 0))),
        compiler_params=pltpu.CompilerParams(
            dimension_semantics=("parallel",),
            vmem_limit_bytes=100 * 1024 * 1024),
    )(xs, wt)
    y = y.reshape(G, _B * Mc, Cout)
    sums = sums.reshape(G, _B, Cout)
    ssqs = ssqs.reshape(G, _B, Cout)

    gamma2 = gamma.reshape(1, Cout)
    beta2 = beta.reshape(1, Cout)
    inv_count = float(1.0 / count)

    def norm_kernel(y_ref, sums_ref, ssqs_ref, gamma_ref, beta_ref, out_ref):
        mean = jnp.sum(sums_ref[...], axis=(0, 1)).reshape(1, -1) * inv_count
        msq = jnp.sum(ssqs_ref[...], axis=(0, 1)).reshape(1, -1) * inv_count
        var = jnp.maximum(msq - mean * mean, 0.0)
        scale = gamma_ref[...] * lax.rsqrt(var + _EPS)
        shift = beta_ref[...] - mean * scale
        z = jnp.maximum(y_ref[0].astype(jnp.float32) * scale + shift, 0.0)
        out_ref[0] = z

    out = pl.pallas_call(
        norm_kernel,
        out_shape=jax.ShapeDtypeStruct((G, _B * Mc, Cout), jnp.float32),
        grid=(G,),
        in_specs=[pl.BlockSpec((1, _B * Mc, Cout), lambda n: (n, 0, 0)),
                  pl.BlockSpec((G, _B, Cout), lambda n: (0, 0, 0)),
                  pl.BlockSpec((G, _B, Cout), lambda n: (0, 0, 0)),
                  pl.BlockSpec((1, Cout), lambda n: (0, 0)),
                  pl.BlockSpec((1, Cout), lambda n: (0, 0))],
        out_specs=pl.BlockSpec((1, _B * Mc, Cout), lambda n: (n, 0, 0)),
        compiler_params=pltpu.CompilerParams(
            dimension_semantics=("parallel",),
            vmem_limit_bytes=100 * 1024 * 1024),
    )(y, sums, ssqs, gamma2, beta2)

    return out.reshape(N, Hout, Wout, Cout).transpose(0, 3, 1, 2)


# speedup vs baseline: 1.1239x; 1.1239x over previous
import jax
import jax.numpy as jnp
from jax import lax
from jax.experimental import pallas as pl
from jax.experimental.pallas import tpu as pltpu

_KS = 4
_ST = 2
_PD = 1
_EPS = 1e-5
_B = 4


def _ceil_to(a, b):
    return (a + b - 1) // b * b


def kernel(x, weight, bias, gamma, beta):
    del bias
    N, C, H, W = x.shape
    Cout = weight.shape[0]
    Hout = (H + 2 * _PD - _KS) // _ST + 1
    Wout = (W + 2 * _PD - _KS) // _ST + 1
    Hc, Wc = Hout + 1, Wout + 1
    M = Hout * Wc
    Mc = Hout * Wout
    C4 = 4 * C
    Mp = _ceil_to(Hc * Wc + 2, 16)
    count = N * Mc
    G = N // _B

    xp = jnp.pad(x.astype(jnp.bfloat16),
                 ((0, 0), (0, 0), (_PD, _PD), (_PD, _PD)))
    xs = xp.reshape(N, C, Hc, 2, Wc, 2).transpose(0, 2, 4, 3, 5, 1)
    xs = xs.reshape(G, _B * Hc * Wc, C4)

    w2 = weight.transpose(1, 2, 3, 0).reshape(C, _KS * _KS, Cout)

    offs = tuple(dh * Wc + dw for dh in range(2) for dw in range(2))

    HW = Hc * Wc

    def conv_kernel(xs_ref, w_ref, y_ref, sum_ref, ssq_ref):
        wb = w_ref[...].astype(jnp.bfloat16)
        wt = jnp.concatenate(
            [jnp.concatenate(
                [wb[:, (2 * dh + ph) * _KS + 2 * dw + pw, :]
                 for ph in range(2) for pw in range(2)], axis=0)
             for dh in range(2) for dw in range(2)], axis=1)
        t = jnp.dot(xs_ref[0], wt, preferred_element_type=jnp.float32)
        t = jnp.pad(t, ((0, 2), (0, 0)))
        s_acc = None
        q_acc = None
        for k in range(_B):
            b = k * HW
            y = (t[b + offs[0]:b + offs[0] + M, :Cout]
                 + t[b + offs[1]:b + offs[1] + M, Cout:2 * Cout]
                 + t[b + offs[2]:b + offs[2] + M, 2 * Cout:3 * Cout]
                 + t[b + offs[3]:b + offs[3] + M, 3 * Cout:])
            yc = jnp.concatenate(
                [y[i * Wc:i * Wc + Wout] for i in range(Hout)], axis=0)
            s = jnp.sum(yc, axis=0, keepdims=True)
            q = jnp.sum(yc * yc, axis=0, keepdims=True)
            s_acc = s if s_acc is None else s_acc + s
            q_acc = q if q_acc is None else q_acc + q
            y_ref[0, k * Mc:(k + 1) * Mc, :] = yc.astype(jnp.bfloat16)
        sum_ref[0] = s_acc
        ssq_ref[0] = q_acc

    y, sums, ssqs = pl.pallas_call(
        conv_kernel,
        out_shape=(jax.ShapeDtypeStruct((G, _B * Mc, Cout), jnp.bfloat16),
                   jax.ShapeDtypeStruct((G, 1, Cout), jnp.float32),
                   jax.ShapeDtypeStruct((G, 1, Cout), jnp.float32)),
        grid=(G,),
        in_specs=[pl.BlockSpec((1, _B * HW, C4), lambda n: (n, 0, 0)),
                  pl.BlockSpec((C, _KS * _KS, Cout), lambda n: (0, 0, 0))],
        out_specs=(pl.BlockSpec((1, _B * Mc, Cout), lambda n: (n, 0, 0)),
                   pl.BlockSpec((1, 1, Cout), lambda n: (n, 0, 0)),
                   pl.BlockSpec((1, 1, Cout), lambda n: (n, 0, 0))),
        compiler_params=pltpu.CompilerParams(
            dimension_semantics=("parallel",),
            vmem_limit_bytes=100 * 1024 * 1024),
    )(xs, w2)

    gamma2 = gamma.reshape(1, Cout)
    beta2 = beta.reshape(1, Cout)
    inv_count = float(1.0 / count)

    def norm_kernel(y_ref, sums_ref, ssqs_ref, gamma_ref, beta_ref, out_ref):
        mean = jnp.sum(sums_ref[:, 0, :], axis=0, keepdims=True) * inv_count
        msq = jnp.sum(ssqs_ref[:, 0, :], axis=0, keepdims=True) * inv_count
        var = jnp.maximum(msq - mean * mean, 0.0)
        scale = gamma_ref[...] * lax.rsqrt(var + _EPS)
        shift = beta_ref[...] - mean * scale
        z = jnp.maximum(y_ref[0].astype(jnp.float32) * scale + shift, 0.0)
        out_ref[0] = z

    out = pl.pallas_call(
        norm_kernel,
        out_shape=jax.ShapeDtypeStruct((G, _B * Mc, Cout), jnp.float32),
        grid=(G,),
        in_specs=[pl.BlockSpec((1, _B * Mc, Cout), lambda n: (n, 0, 0)),
                  pl.BlockSpec((G, 1, Cout), lambda n: (0, 0, 0)),
                  pl.BlockSpec((G, 1, Cout), lambda n: (0, 0, 0)),
                  pl.BlockSpec((1, Cout), lambda n: (0, 0)),
                  pl.BlockSpec((1, Cout), lambda n: (0, 0))],
        out_specs=pl.BlockSpec((1, _B * Mc, Cout), lambda n: (n, 0, 0)),
        compiler_params=pltpu.CompilerParams(
            dimension_semantics=("parallel",),
            vmem_limit_bytes=100 * 1024 * 1024),
    )(y, sums, ssqs, gamma2, beta2)

    return out.reshape(N, Hout, Wout, Cout).transpose(0, 3, 1, 2)
